# Initial kernel scaffold; baseline (speedup 1.0000x reference)
import jax
import jax.numpy as jnp
from jax.experimental import pallas as pl


def kernel(x, edge_index, enc_W1, enc_b1, enc_W2, enc_b2, attr_W1, attr_b1, attr_W2, attr_b2, str_W1, str_b1):
    raise NotImplementedError("write your pallas kernel here")


# speedup vs baseline: 5.8171x; 5.8171x over previous
import functools

import jax
import jax.numpy as jnp
from jax import lax
from jax.experimental import pallas as pl
from jax.experimental.pallas import tpu as pltpu
from jax.experimental.pallas import tpu_sc as plsc

N = 10000
E = 160000
D = 256
HALF = 128

NP = 10240
NTILES = 16
NC = 2
RPT = NP // NTILES
CH = 128
NCHUNK = NP // CH
EPT_PAD = NCHUNK * CH
NROW = NP // 128
ED_CH = 40
ED_PAD = ED_CH * CH

BR = 1024
GR = 2000
GC = 1280


def _deg_body(dst_hbm, out_hbm, dst_v, hist, sem):
    cid = lax.axis_index("c")
    sid = lax.axis_index("s")
    wid = sid * NC + cid
    pltpu.async_copy(dst_hbm.at[wid], dst_v, sem).wait()
    zeros = jnp.zeros((16,), jnp.float32)

    @pl.loop(0, NROW)
    def _(i):
        @pl.loop(0, 8)
        def _(j):
            hist[i, pl.ds(j * 16, 16)] = zeros

    ones = jnp.ones((16,), jnp.float32)

    @pl.loop(0, ED_CH)
    def _(i):
        @pl.loop(0, 8)
        def _(j):
            idx = dst_v[i, pl.ds(j * 16, 16)]
            plsc.addupdate_scatter(hist, [idx >> 7, idx & 127], ones)

    pltpu.sync_copy(hist, out_hbm.at[wid])


def _make_deg_kernel():
    mesh = plsc.VectorSubcoreMesh(core_axis_name="c", subcore_axis_name="s")
    return pl.kernel(
        _deg_body,
        out_type=jax.ShapeDtypeStruct((NC * NTILES, NROW, 128), jnp.float32),
        mesh=mesh,
        compiler_params=pltpu.CompilerParams(needs_layout_passes=False),
        scratch_types=[
            pltpu.VMEM((ED_CH, CH), jnp.int32),
            pltpu.VMEM((NROW, 128), jnp.float32),
            pltpu.SemaphoreType.DMA,
        ],
    )


def _conv_body(y2d, src_hbm, dst_hbm, out_hbm, src_v, dst_v, rows, acc,
               sem_i, sem_g):
    cid = lax.axis_index("c")
    sid = lax.axis_index("s")

    pltpu.async_copy(src_hbm.at[sid], src_v, sem_i).wait()
    pltpu.async_copy(dst_hbm.at[sid], dst_v, sem_i).wait()
    base = sid * RPT
    pltpu.sync_copy(y2d.at[pl.ds(cid * NP + base, RPT)],
                    acc.at[pl.ds(base, RPT)])

    off = jnp.broadcast_to(cid * NP, (16,)).astype(jnp.int32)

    @pl.loop(0, NCHUNK)
    def _(i):
        @pl.loop(0, 8)
        def _(j):
            src_v[i, pl.ds(j * 16, 16)] = src_v[i, pl.ds(j * 16, 16)] + off

    plsc.subcore_barrier()

    @pl.loop(0, NCHUNK)
    def _(j):
        pltpu.async_copy(y2d.at[src_v.at[j]], rows, sem_g).wait()
        pltpu.sync_copy(rows, acc.at[dst_v.at[j]], add=True)

    plsc.subcore_barrier()
    pltpu.sync_copy(acc.at[pl.ds(base, RPT)],
                    out_hbm.at[pl.ds(cid * NP + base, RPT)])


def _make_conv_kernel():
    mesh = plsc.VectorSubcoreMesh(core_axis_name="c", subcore_axis_name="s")
    return pl.kernel(
        _conv_body,
        out_type=jax.ShapeDtypeStruct((NC * NP, HALF), jnp.float32),
        mesh=mesh,
        compiler_params=pltpu.CompilerParams(needs_layout_passes=False),
        scratch_types=[
            pltpu.VMEM((NCHUNK, CH), jnp.int32),
            pltpu.VMEM((NCHUNK, CH), jnp.int32),
            pltpu.VMEM((CH, HALF), jnp.float32),
            pltpu.VMEM_SHARED((NP, HALF), jnp.float32),
            pltpu.SemaphoreType.DMA,
            pltpu.SemaphoreType.DMA,
        ],
    )


def _split(y):
    return jnp.stack([y[:, :HALF], y[:, HALF:]], axis=0)


def _prep_body(hist_ref, x_ref, w_ref, dinvb_ref, y_ref):
    deg = jnp.sum(hist_ref[...], axis=1, keepdims=True) + 1.0
    dvb = jnp.broadcast_to(lax.rsqrt(deg), (BR, D))
    dinvb_ref[...] = dvb
    y = jnp.dot(x_ref[...], w_ref[...], preferred_element_type=jnp.float32)
    y_ref[...] = _split(y * dvb)


def _prep_call(hist, x, w1):
    return pl.pallas_call(
        _prep_body,
        grid=(NP // BR,),
        in_specs=[
            pl.BlockSpec((BR, NC * NTILES), lambda i: (i, 0)),
            pl.BlockSpec((BR, D), lambda i: (i, 0)),
            pl.BlockSpec((D, D), lambda i: (0, 0)),
        ],
        out_specs=[
            pl.BlockSpec((BR, D), lambda i: (i, 0)),
            pl.BlockSpec((2, BR, HALF), lambda i: (0, i, 0)),
        ],
        out_shape=[
            jax.ShapeDtypeStruct((NP, D), jnp.float32),
            jax.ShapeDtypeStruct((2, NP, HALF), jnp.float32),
        ],
    )(hist, x, w1)


def _conv_tc_body(relu, nw, acc_ref, dinvb_ref, b_ref, *w_and_out):
    w_refs = w_and_out[:nw]
    out_refs = w_and_out[nw:]
    accb = acc_ref[...]
    z = jnp.concatenate([accb[0], accb[1]], axis=1)
    dvb = dinvb_ref[...]
    z = z * dvb + b_ref[...]
    if relu:
        z = jnp.maximum(z, 0.0)
    for w_ref, out_ref in zip(w_refs, out_refs):
        y = jnp.dot(z, w_ref[...], preferred_element_type=jnp.float32)
        out_ref[...] = _split(y * dvb)


def _conv_tc_call(acc, dinvb, b, ws, relu):
    nw = len(ws)
    return pl.pallas_call(
        functools.partial(_conv_tc_body, relu, nw),
        grid=(NP // BR,),
        in_specs=[
            pl.BlockSpec((2, BR, HALF), lambda i: (0, i, 0)),
            pl.BlockSpec((BR, D), lambda i: (i, 0)),
            pl.BlockSpec((1, D), lambda i: (0, 0)),
        ] + [pl.BlockSpec((D, D), lambda i: (0, 0))] * nw,
        out_specs=[pl.BlockSpec((2, BR, HALF), lambda i: (0, i, 0))] * nw,
        out_shape=[jax.ShapeDtypeStruct((2, NP, HALF), jnp.float32)] * nw,
    )(acc, dinvb, b, *ws)


def _final_body(acc4_ref, acc5_ref, dinvb_ref, ba_ref, bs_ref,
                x_ref, h_ref):
    dvb = dinvb_ref[...]
    a4 = acc4_ref[...]
    x_ref[...] = jnp.concatenate([a4[0], a4[1]], axis=1) * dvb + ba_ref[...]
    a5 = acc5_ref[...]
    h_ref[...] = jnp.concatenate([a5[0], a5[1]], axis=1) * dvb + bs_ref[...]


def _final_call(acc4, acc5, dinvb, ba, bs):
    return pl.pallas_call(
        _final_body,
        grid=(NP // BR,),
        in_specs=[
            pl.BlockSpec((2, BR, HALF), lambda i: (0, i, 0)),
            pl.BlockSpec((2, BR, HALF), lambda i: (0, i, 0)),
            pl.BlockSpec((BR, D), lambda i: (i, 0)),
            pl.BlockSpec((1, D), lambda i: (0, 0)),
            pl.BlockSpec((1, D), lambda i: (0, 0)),
        ],
        out_specs=[
            pl.BlockSpec((BR, D), lambda i: (i, 0)),
            pl.BlockSpec((BR, D), lambda i: (i, 0)),
        ],
        out_shape=[
            jax.ShapeDtypeStruct((NP, D), jnp.float32),
            jax.ShapeDtypeStruct((NP, D), jnp.float32),
        ],
    )(acc4, acc5, dinvb, ba, bs)


def _gram_body(a_ref, b_ref, out_ref):
    out_ref[0] = lax.dot_general(
        a_ref[...], b_ref[...], (((1,), (1,)), ((), ())),
        preferred_element_type=jnp.float32)


def _gram_call(h):
    out = pl.pallas_call(
        _gram_body,
        grid=(N // GR, pl.cdiv(N, GC)),
        in_specs=[
            pl.BlockSpec((GR, D), lambda i, j: (i, 0)),
            pl.BlockSpec((GC, D), lambda i, j: (j, 0)),
        ],
        out_specs=pl.BlockSpec((1, GR, GC), lambda i, j: (i, 0, j)),
        out_shape=jax.ShapeDtypeStruct((N // GR, GR, N), jnp.float32),
    )(h, h)
    return out.reshape(N, N)


def _pad_edges(idx, tiles, chunks, fill):
    per = chunks * CH
    take = E // tiles
    t = idx.reshape(tiles, take)
    pad = jnp.full((tiles, per - take), fill, jnp.int32)
    return jnp.concatenate([t, pad], axis=1).reshape(tiles, chunks, CH)


def kernel(x, edge_index, enc_W1, enc_b1, enc_W2, enc_b2,
           attr_W1, attr_b1, attr_W2, attr_b2, str_W1, str_b1):
    src = edge_index[0].astype(jnp.int32)
    dst = edge_index[1].astype(jnp.int32)

    src_p = _pad_edges(src, NTILES, NCHUNK, 0)
    dst_p = _pad_edges(dst, NTILES, NCHUNK, N)
    dst_d = _pad_edges(dst, NC * NTILES, ED_CH, N)
    x_p = jnp.concatenate([x, jnp.zeros((NP - N, D), jnp.float32)], axis=0)

    deg_k = _make_deg_kernel()
    conv_k = _make_conv_kernel()

    hist = deg_k(dst_d)
    hist_n = hist.reshape(NC * NTILES, NP).T
    dinvb, y1 = _prep_call(hist_n, x_p, enc_W1)

    def conv(y):
        acc = conv_k(y.reshape(NC * NP, HALF), src_p, dst_p)
        return acc.reshape(2, NP, HALF)

    b = lambda v: v.reshape(1, D)

    acc1 = conv(y1)
    (y2,) = _conv_tc_call(acc1, dinvb, b(enc_b1), [enc_W2], relu=True)
    acc2 = conv(y2)
    y3, y5 = _conv_tc_call(acc2, dinvb, b(enc_b2), [attr_W1, str_W1],
                           relu=False)
    acc3 = conv(y3)
    (y4,) = _conv_tc_call(acc3, dinvb, b(attr_b1), [attr_W2], relu=True)
    acc4 = conv(y4)
    acc5 = conv(y5)
    x_full, h_full = _final_call(acc4, acc5, dinvb, b(attr_b2), b(str_b1))
    x_ = x_full[:N]
    s_ = _gram_call(h_full[:N])
    return (x_, s_)


# speedup vs baseline: 7.0887x; 1.2186x over previous
import functools

import jax
import jax.numpy as jnp
from jax import lax
from jax.experimental import pallas as pl
from jax.experimental.pallas import tpu as pltpu
from jax.experimental.pallas import tpu_sc as plsc

N = 10000
E = 160000
D = 256
HALF = 128

NP = 10240
NTILES = 16
NC = 2
RPT = NP // NTILES
CH = 128
NCHUNK = NP // CH
EPT_PAD = NCHUNK * CH
NROW = NP // 128
ED_CH = 40
ED_PAD = ED_CH * CH

BR = 1024
GR = 2000
GC = 1280


def _deg_body(dst_hbm, out_hbm, dst_v, hist, sem):
    cid = lax.axis_index("c")
    sid = lax.axis_index("s")
    wid = sid * NC + cid
    pltpu.async_copy(dst_hbm.at[wid], dst_v, sem).wait()
    zeros = jnp.zeros((16,), jnp.float32)

    @pl.loop(0, NROW)
    def _(i):
        @pl.loop(0, 8)
        def _(j):
            hist[i, pl.ds(j * 16, 16)] = zeros

    ones = jnp.ones((16,), jnp.float32)

    @pl.loop(0, ED_CH)
    def _(i):
        @pl.loop(0, 8)
        def _(j):
            idx = dst_v[i, pl.ds(j * 16, 16)]
            plsc.addupdate_scatter(hist, [idx >> 7, idx & 127], ones)

    pltpu.sync_copy(hist, out_hbm.at[wid])


def _make_deg_kernel():
    mesh = plsc.VectorSubcoreMesh(core_axis_name="c", subcore_axis_name="s")
    return pl.kernel(
        _deg_body,
        out_type=jax.ShapeDtypeStruct((NC * NTILES, NROW, 128), jnp.float32),
        mesh=mesh,
        compiler_params=pltpu.CompilerParams(needs_layout_passes=False),
        scratch_types=[
            pltpu.VMEM((ED_CH, CH), jnp.int32),
            pltpu.VMEM((NROW, 128), jnp.float32),
            pltpu.SemaphoreType.DMA,
        ],
    )


G = 16
NGRP = NCHUNK // G


def _conv_body(y2d, src_hbm, dst_hbm, out_hbm, src_v, dst_v, rows,
               acc, sem_i, gs0, gs1, ss0, ss1):
    gsem = (gs0, gs1)
    ssem = (ss0, ss1)
    cid = lax.axis_index("c")
    sid = lax.axis_index("s")
    base = sid * RPT
    off = jnp.broadcast_to(cid * NP, (16,)).astype(jnp.int32)

    def idx_load(grp, wait):
        p = grp & 1
        if wait is None:
            pltpu.async_copy(src_hbm.at[sid, pl.ds(grp * G, G)],
                             src_v.at[p], sem_i)
            pltpu.async_copy(dst_hbm.at[sid, pl.ds(grp * G, G)],
                             dst_v.at[p], sem_i)
        else:
            pltpu.make_async_copy(src_hbm.at[sid, pl.ds(grp * G, G)],
                                  src_v.at[p], sem_i).wait()
            pltpu.make_async_copy(dst_hbm.at[sid, pl.ds(grp * G, G)],
                                  dst_v.at[p], sem_i).wait()

    def idx_fix(grp):
        p = grp & 1

        @pl.loop(0, G)
        def _(i):
            @pl.loop(0, 8)
            def _(j):
                sl = pl.ds(j * 16, 16)
                src_v[p, i, sl] = src_v[p, i, sl] + off

    def gather(k):
        grp, j = divmod(k, G)
        pltpu.async_copy(y2d.at[src_v.at[grp & 1, j]], rows.at[k & 1],
                         gsem[k & 1])

    def gwait(k):
        grp, j = divmod(k, G)
        pltpu.make_async_copy(y2d.at[src_v.at[grp & 1, j]], rows.at[k & 1],
                              gsem[k & 1]).wait()

    def scatter(k):
        grp, j = divmod(k, G)
        pltpu.async_copy(rows.at[k & 1], acc.at[dst_v.at[grp & 1, j]],
                         ssem[k & 1], add=True)

    def swait(k):
        grp, j = divmod(k, G)
        pltpu.make_async_copy(rows.at[k & 1], acc.at[dst_v.at[grp & 1, j]],
                              ssem[k & 1]).wait()

    idx_load(0, None)
    pltpu.sync_copy(y2d.at[pl.ds(cid * NP + base, RPT)],
                    acc.at[pl.ds(base, RPT)])
    idx_load(0, True)
    idx_fix(0)
    plsc.subcore_barrier()

    for k in range(NCHUNK):
        grp, j = divmod(k, G)
        if k >= 2:
            swait(k - 2)
        gather(k)
        if j == 4 and grp + 1 < NGRP:
            idx_load(grp + 1, None)
        if j == G - 1 and grp + 1 < NGRP:
            idx_load(grp + 1, True)
            idx_fix(grp + 1)
        if k >= 1:
            gwait(k - 1)
            scatter(k - 1)
    gwait(NCHUNK - 1)
    scatter(NCHUNK - 1)
    swait(NCHUNK - 2)
    swait(NCHUNK - 1)

    plsc.subcore_barrier()
    pltpu.sync_copy(acc.at[pl.ds(base, RPT)],
                    out_hbm.at[pl.ds(cid * NP + base, RPT)])


def _make_conv_kernel():
    mesh = plsc.VectorSubcoreMesh(core_axis_name="c", subcore_axis_name="s")
    return pl.kernel(
        _conv_body,
        out_type=jax.ShapeDtypeStruct((NC * NP, HALF), jnp.float32),
        mesh=mesh,
        compiler_params=pltpu.CompilerParams(needs_layout_passes=False),
        scratch_types=[
            pltpu.VMEM((2, G, CH), jnp.int32),
            pltpu.VMEM((2, G, CH), jnp.int32),
            pltpu.VMEM((2, CH, HALF), jnp.float32),
            pltpu.VMEM_SHARED((NP, HALF), jnp.float32),
            pltpu.SemaphoreType.DMA,
            pltpu.SemaphoreType.DMA,
            pltpu.SemaphoreType.DMA,
            pltpu.SemaphoreType.DMA,
            pltpu.SemaphoreType.DMA,
        ],
    )


def _split(y):
    return jnp.stack([y[:, :HALF], y[:, HALF:]], axis=0)


def _prep_body(hist_ref, x_ref, w_ref, dinvb_ref, y_ref):
    deg = jnp.sum(hist_ref[...], axis=1, keepdims=True) + 1.0
    dvb = jnp.broadcast_to(lax.rsqrt(deg), (BR, D))
    dinvb_ref[...] = dvb
    y = jnp.dot(x_ref[...], w_ref[...], preferred_element_type=jnp.float32)
    y_ref[...] = _split(y * dvb)


def _prep_call(hist, x, w1):
    return pl.pallas_call(
        _prep_body,
        grid=(NP // BR,),
        in_specs=[
            pl.BlockSpec((BR, NC * NTILES), lambda i: (i, 0)),
            pl.BlockSpec((BR, D), lambda i: (i, 0)),
            pl.BlockSpec((D, D), lambda i: (0, 0)),
        ],
        out_specs=[
            pl.BlockSpec((BR, D), lambda i: (i, 0)),
            pl.BlockSpec((2, BR, HALF), lambda i: (0, i, 0)),
        ],
        out_shape=[
            jax.ShapeDtypeStruct((NP, D), jnp.float32),
            jax.ShapeDtypeStruct((2, NP, HALF), jnp.float32),
        ],
    )(hist, x, w1)


def _conv_tc_body(relu, nw, acc_ref, dinvb_ref, b_ref, *w_and_out):
    w_refs = w_and_out[:nw]
    out_refs = w_and_out[nw:]
    accb = acc_ref[...]
    z = jnp.concatenate([accb[0], accb[1]], axis=1)
    dvb = dinvb_ref[...]
    z = z * dvb + b_ref[...]
    if relu:
        z = jnp.maximum(z, 0.0)
    for w_ref, out_ref in zip(w_refs, out_refs):
        y = jnp.dot(z, w_ref[...], preferred_element_type=jnp.float32)
        out_ref[...] = _split(y * dvb)


def _conv_tc_call(acc, dinvb, b, ws, relu):
    nw = len(ws)
    return pl.pallas_call(
        functools.partial(_conv_tc_body, relu, nw),
        grid=(NP // BR,),
        in_specs=[
            pl.BlockSpec((2, BR, HALF), lambda i: (0, i, 0)),
            pl.BlockSpec((BR, D), lambda i: (i, 0)),
            pl.BlockSpec((1, D), lambda i: (0, 0)),
        ] + [pl.BlockSpec((D, D), lambda i: (0, 0))] * nw,
        out_specs=[pl.BlockSpec((2, BR, HALF), lambda i: (0, i, 0))] * nw,
        out_shape=[jax.ShapeDtypeStruct((2, NP, HALF), jnp.float32)] * nw,
    )(acc, dinvb, b, *ws)


def _final_body(acc4_ref, acc5_ref, dinvb_ref, ba_ref, bs_ref,
                x_ref, h_ref):
    dvb = dinvb_ref[...]
    a4 = acc4_ref[...]
    x_ref[...] = jnp.concatenate([a4[0], a4[1]], axis=1) * dvb + ba_ref[...]
    a5 = acc5_ref[...]
    h_ref[...] = jnp.concatenate([a5[0], a5[1]], axis=1) * dvb + bs_ref[...]


def _final_call(acc4, acc5, dinvb, ba, bs):
    return pl.pallas_call(
        _final_body,
        grid=(NP // BR,),
        in_specs=[
            pl.BlockSpec((2, BR, HALF), lambda i: (0, i, 0)),
            pl.BlockSpec((2, BR, HALF), lambda i: (0, i, 0)),
            pl.BlockSpec((BR, D), lambda i: (i, 0)),
            pl.BlockSpec((1, D), lambda i: (0, 0)),
            pl.BlockSpec((1, D), lambda i: (0, 0)),
        ],
        out_specs=[
            pl.BlockSpec((BR, D), lambda i: (i, 0)),
            pl.BlockSpec((BR, D), lambda i: (i, 0)),
        ],
        out_shape=[
            jax.ShapeDtypeStruct((NP, D), jnp.float32),
            jax.ShapeDtypeStruct((NP, D), jnp.float32),
        ],
    )(acc4, acc5, dinvb, ba, bs)


def _gram_body(a_ref, b_ref, out_ref):
    out_ref[0] = lax.dot_general(
        a_ref[...], b_ref[...], (((1,), (1,)), ((), ())),
        preferred_element_type=jnp.float32)


def _gram_call(h):
    out = pl.pallas_call(
        _gram_body,
        grid=(N // GR, pl.cdiv(N, GC)),
        in_specs=[
            pl.BlockSpec((GR, D), lambda i, j: (i, 0)),
            pl.BlockSpec((GC, D), lambda i, j: (j, 0)),
        ],
        out_specs=pl.BlockSpec((1, GR, GC), lambda i, j: (i, 0, j)),
        out_shape=jax.ShapeDtypeStruct((N // GR, GR, N), jnp.float32),
    )(h, h)
    return out.reshape(N, N)


def _pad_edges(idx, tiles, chunks, fill):
    per = chunks * CH
    take = E // tiles
    t = idx.reshape(tiles, take)
    pad = jnp.full((tiles, per - take), fill, jnp.int32)
    return jnp.concatenate([t, pad], axis=1).reshape(tiles, chunks, CH)


def kernel(x, edge_index, enc_W1, enc_b1, enc_W2, enc_b2,
           attr_W1, attr_b1, attr_W2, attr_b2, str_W1, str_b1):
    src = edge_index[0].astype(jnp.int32)
    dst = edge_index[1].astype(jnp.int32)

    src_p = _pad_edges(src, NTILES, NCHUNK, 0)
    dst_p = _pad_edges(dst, NTILES, NCHUNK, N)
    dst_d = _pad_edges(dst, NC * NTILES, ED_CH, N)
    x_p = jnp.concatenate([x, jnp.zeros((NP - N, D), jnp.float32)], axis=0)

    deg_k = _make_deg_kernel()
    conv_k = _make_conv_kernel()

    hist = deg_k(dst_d)
    hist_n = hist.reshape(NC * NTILES, NP).T
    dinvb, y1 = _prep_call(hist_n, x_p, enc_W1)

    def conv(y):
        acc = conv_k(y.reshape(NC * NP, HALF), src_p, dst_p)
        return acc.reshape(2, NP, HALF)

    b = lambda v: v.reshape(1, D)

    acc1 = conv(y1)
    (y2,) = _conv_tc_call(acc1, dinvb, b(enc_b1), [enc_W2], relu=True)
    acc2 = conv(y2)
    y3, y5 = _conv_tc_call(acc2, dinvb, b(enc_b2), [attr_W1, str_W1],
                           relu=False)
    acc3 = conv(y3)
    (y4,) = _conv_tc_call(acc3, dinvb, b(attr_b1), [attr_W2], relu=True)
    acc4 = conv(y4)
    acc5 = conv(y5)
    x_full, h_full = _final_call(acc4, acc5, dinvb, b(attr_b2), b(str_b1))
    x_ = x_full[:N]
    s_ = _gram_call(h_full[:N])
    return (x_, s_)
